# baseline (speedup 1.0000x reference)
import functools

import jax
import jax.numpy as jnp
from jax.experimental import pallas as pl
from jax.experimental.pallas import tpu as pltpu

_BATCH = 16384
_DIM = 2048
_G = _DIM // 128
_BLK = 256


def _tc_shuffle(x, perm2d):
  grid = (_BATCH // _BLK,)

  def body(x_ref, perm_ref, out_ref):
    p = perm_ref[...]
    lane = jax.lax.rem(p, jnp.int32(128))
    grp = jax.lax.div(p, jnp.int32(128))
    for g in range(_G):
      lane_g = jnp.broadcast_to(lane[:, g * 128:(g + 1) * 128], (_BLK, 128))
      grp_g = grp[:, g * 128:(g + 1) * 128]
      acc = jnp.zeros((_BLK, 128), jnp.float32)
      for h in range(_G):
        src = x_ref[:, h * 128:(h + 1) * 128]
        gathered = jnp.take_along_axis(src, lane_g, axis=1)
        acc = jnp.where(grp_g == h, gathered, acc)
      out_ref[:, g * 128:(g + 1) * 128] = acc

  return pl.pallas_call(
      body,
      grid=grid,
      in_specs=[
          pl.BlockSpec((_BLK, _DIM), lambda i: (i, 0)),
          pl.BlockSpec((1, _DIM), lambda i: (0, 0)),
      ],
      out_specs=pl.BlockSpec((_BLK, _DIM), lambda i: (i, 0)),
      out_shape=jax.ShapeDtypeStruct((_BATCH, _DIM), jnp.float32),
  )(x, perm2d)


def kernel(x, permutation):
  perm2d = permutation.astype(jnp.int32).reshape(1, _DIM)
  return _tc_shuffle(x, perm2d)


# speedup vs baseline: 4.5287x; 4.5287x over previous
import functools

import jax
import jax.numpy as jnp
from jax import lax
from jax.experimental import pallas as pl
from jax.experimental.pallas import tpu as pltpu
from jax.experimental.pallas import tpu_sc as plsc

_NC = 2
_NS = 16
_L = 16

_BATCH = 16384
_DIM = 2048

_SC_ROWS = 14336
_TC_ROWS = _BATCH - _SC_ROWS

_NW = _NC * _NS
_ROWS_PER_W = _SC_ROWS // _NW
_CHUNK = 8
_NCHUNKS = _ROWS_PER_W // _CHUNK
_NBUF = 3
_NGRP = (_NCHUNKS - _NBUF) // _NBUF
_NPEEL = _NCHUNKS - _NBUF * (1 + _NGRP)
_JVECS = _DIM // _L

_G = _DIM // 128
_BLK = 256


def _sc_shuffle(x, perm):
  mesh = plsc.VectorSubcoreMesh(core_axis_name="c", subcore_axis_name="s")

  @functools.partial(
      pl.kernel,
      out_type=jax.ShapeDtypeStruct((_BATCH, _DIM), jnp.float32),
      mesh=mesh,
      scratch_types=[
          pltpu.VMEM((_DIM,), jnp.int32),
          [pltpu.VMEM((_CHUNK, _DIM), jnp.float32) for _ in range(_NBUF)],
          [pltpu.VMEM((_CHUNK, _DIM), jnp.float32) for _ in range(_NBUF)],
          [pltpu.SemaphoreType.DMA for _ in range(_NBUF)],
          [pltpu.SemaphoreType.DMA for _ in range(_NBUF)],
      ],
      compiler_params=pltpu.CompilerParams(needs_layout_passes=False),
  )
  def k(x_hbm, perm_hbm, out_hbm, perm_v, in_v, out_v, in_sem, out_sem):
    wid = lax.axis_index("s") * _NC + lax.axis_index("c")
    pltpu.sync_copy(perm_hbm, perm_v)
    row0 = wid * _ROWS_PER_W

    def in_slice(c):
      return x_hbm.at[pl.ds(row0 + c * _CHUNK, _CHUNK), :]

    def out_slice(c):
      return out_hbm.at[pl.ds(row0 + c * _CHUNK, _CHUNK), :]

    def process(c, b, first_round):
      pltpu.make_async_copy(in_slice(c), in_v[b], in_sem[b]).wait()
      if not first_round:
        pltpu.make_async_copy(out_v[b], out_slice(c), out_sem[b]).wait()

      @plsc.parallel_loop(0, _JVECS, unroll=4)
      def j_body(j):
        idx = perm_v[pl.ds(j * _L, _L)]
        for r in range(_CHUNK):
          row_idx = jnp.full((_L,), r, jnp.int32)
          vals = plsc.load_gather(in_v[b], [row_idx, idx])
          out_v[b][r, pl.ds(j * _L, _L)] = vals

      pltpu.async_copy(out_v[b], out_slice(c), out_sem[b])

      @pl.when(c + _NBUF < _NCHUNKS)
      def _():
        pltpu.async_copy(in_slice(c + _NBUF), in_v[b], in_sem[b])

    for b in range(_NBUF):
      pltpu.async_copy(in_slice(b), in_v[b], in_sem[b])

    for b in range(_NBUF):
      process(b, b, True)

    def grp_body(g, carry):
      for b in range(_NBUF):
        c = (g + 1) * _NBUF + b
        process(c, b, False)
      return carry

    lax.fori_loop(0, _NGRP, grp_body, 0)

    for p in range(_NPEEL):
      c = _NBUF * (1 + _NGRP) + p
      process(c, c % _NBUF, False)

    for c in range(_NCHUNKS - _NBUF, _NCHUNKS):
      b = c % _NBUF
      pltpu.make_async_copy(out_v[b], out_slice(c), out_sem[b]).wait()

  return k(x, perm)


def _tc_shuffle(x, perm2d):
  grid = (_TC_ROWS // _BLK,)
  row_blk0 = _SC_ROWS // _BLK

  def body(x_ref, perm_ref, out_ref):
    p = perm_ref[...]
    lane = lax.rem(p, jnp.int32(128))
    grp = lax.div(p, jnp.int32(128))
    for g in range(_G):
      lane_g = jnp.broadcast_to(lane[:, g * 128:(g + 1) * 128], (_BLK, 128))
      grp_g = grp[:, g * 128:(g + 1) * 128]
      acc = jnp.zeros((_BLK, 128), jnp.float32)
      for h in range(_G):
        src = x_ref[:, h * 128:(h + 1) * 128]
        gathered = jnp.take_along_axis(src, lane_g, axis=1)
        acc = jnp.where(grp_g == h, gathered, acc)
      out_ref[:, g * 128:(g + 1) * 128] = acc

  return pl.pallas_call(
      body,
      grid=grid,
      in_specs=[
          pl.BlockSpec((_BLK, _DIM), lambda i: (row_blk0 + i, 0)),
          pl.BlockSpec((1, _DIM), lambda i: (0, 0)),
      ],
      out_specs=pl.BlockSpec((_BLK, _DIM), lambda i: (i, 0)),
      out_shape=jax.ShapeDtypeStruct((_TC_ROWS, _DIM), jnp.float32),
  )(x, perm2d)


def kernel(x, permutation):
  perm = permutation.astype(jnp.int32)
  sc_out = _sc_shuffle(x, perm)
  tc_out = _tc_shuffle(x, perm.reshape(1, _DIM))
  return lax.dynamic_update_slice(sc_out, tc_out, (_SC_ROWS, 0))


# speedup vs baseline: 4.9808x; 1.0998x over previous
import functools

import jax
import jax.numpy as jnp
from jax import lax
from jax.experimental import pallas as pl
from jax.experimental.pallas import tpu as pltpu
from jax.experimental.pallas import tpu_sc as plsc

_NC = 2
_NS = 16
_L = 16

_BATCH = 16384
_DIM = 2048
_NW = _NC * _NS
_ROWS_PER_W = _BATCH // _NW
_CHUNK = 4
_NCHUNKS = _ROWS_PER_W // _CHUNK
_NBUF = 7
_NGRP = (_NCHUNKS - _NBUF) // _NBUF
_NPEEL = _NCHUNKS - _NBUF * (1 + _NGRP)
_JVECS = _DIM // _L


def _sc_shuffle(x, perm):
  mesh = plsc.VectorSubcoreMesh(core_axis_name="c", subcore_axis_name="s")

  @functools.partial(
      pl.kernel,
      out_type=jax.ShapeDtypeStruct((_BATCH, _DIM), jnp.float32),
      mesh=mesh,
      scratch_types=[
          pltpu.VMEM((_DIM,), jnp.int32),
          [pltpu.VMEM((_CHUNK, _DIM), jnp.float32) for _ in range(_NBUF)],
          [pltpu.VMEM((_CHUNK, _DIM), jnp.float32) for _ in range(_NBUF)],
          [pltpu.SemaphoreType.DMA for _ in range(_NBUF)],
          [pltpu.SemaphoreType.DMA for _ in range(_NBUF)],
      ],
      compiler_params=pltpu.CompilerParams(needs_layout_passes=False),
  )
  def k(x_hbm, perm_hbm, out_hbm, perm_v, in_v, out_v, in_sem, out_sem):
    wid = lax.axis_index("s") * _NC + lax.axis_index("c")
    pltpu.sync_copy(perm_hbm, perm_v)
    row0 = wid * _ROWS_PER_W

    def in_slice(c):
      return x_hbm.at[pl.ds(row0 + c * _CHUNK, _CHUNK), :]

    def out_slice(c):
      return out_hbm.at[pl.ds(row0 + c * _CHUNK, _CHUNK), :]

    def process(c, b, first_round):
      pltpu.make_async_copy(in_slice(c), in_v[b], in_sem[b]).wait()
      if not first_round:
        pltpu.make_async_copy(out_v[b], out_slice(c), out_sem[b]).wait()

      @plsc.parallel_loop(0, _JVECS, unroll=4)
      def j_body(j):
        idx = perm_v[pl.ds(j * _L, _L)]
        for r in range(_CHUNK):
          row_idx = jnp.full((_L,), r, jnp.int32)
          vals = plsc.load_gather(in_v[b], [row_idx, idx])
          out_v[b][r, pl.ds(j * _L, _L)] = vals

      pltpu.async_copy(out_v[b], out_slice(c), out_sem[b])

      @pl.when(c + _NBUF < _NCHUNKS)
      def _():
        pltpu.async_copy(in_slice(c + _NBUF), in_v[b], in_sem[b])

    for b in range(_NBUF):
      pltpu.async_copy(in_slice(b), in_v[b], in_sem[b])

    for b in range(_NBUF):
      process(b, b, True)

    def grp_body(g, carry):
      for b in range(_NBUF):
        c = (g + 1) * _NBUF + b
        process(c, b, False)
      return carry

    lax.fori_loop(0, _NGRP, grp_body, 0)

    for p in range(_NPEEL):
      c = _NBUF * (1 + _NGRP) + p
      process(c, c % _NBUF, False)

    for c in range(_NCHUNKS - _NBUF, _NCHUNKS):
      b = c % _NBUF
      pltpu.make_async_copy(out_v[b], out_slice(c), out_sem[b]).wait()

  return k(x, perm)


def kernel(x, permutation):
  return _sc_shuffle(x, permutation.astype(jnp.int32))


# speedup vs baseline: 5.0154x; 1.0069x over previous
import functools

import jax
import jax.numpy as jnp
from jax import lax
from jax.experimental import pallas as pl
from jax.experimental.pallas import tpu as pltpu
from jax.experimental.pallas import tpu_sc as plsc

_NC = 2
_NS = 16
_L = 16

_BATCH = 16384
_DIM = 2048
_NW = _NC * _NS
_ROWS_PER_W = _BATCH // _NW
_CHUNK = 8
_NCHUNKS = _ROWS_PER_W // _CHUNK
_NBUF = 3
_NGRP = (_NCHUNKS - _NBUF) // _NBUF
_NPEEL = _NCHUNKS - _NBUF * (1 + _NGRP)
_JVECS = _DIM // _L


def _sc_shuffle(x, perm):
  mesh = plsc.VectorSubcoreMesh(core_axis_name="c", subcore_axis_name="s")

  @functools.partial(
      pl.kernel,
      out_type=jax.ShapeDtypeStruct((_BATCH, _DIM), jnp.float32),
      mesh=mesh,
      scratch_types=[
          pltpu.VMEM((_DIM,), jnp.int32),
          [pltpu.VMEM((_CHUNK, _DIM), jnp.float32) for _ in range(_NBUF)],
          [pltpu.VMEM((_CHUNK, _DIM), jnp.float32) for _ in range(_NBUF)],
          [pltpu.SemaphoreType.DMA for _ in range(_NBUF)],
          [pltpu.SemaphoreType.DMA for _ in range(_NBUF)],
      ],
      compiler_params=pltpu.CompilerParams(needs_layout_passes=False),
  )
  def k(x_hbm, perm_hbm, out_hbm, perm_v, in_v, out_v, in_sem, out_sem):
    wid = lax.axis_index("s") * _NC + lax.axis_index("c")
    pltpu.sync_copy(perm_hbm, perm_v)
    row0 = wid * _ROWS_PER_W

    def in_slice(c):
      return x_hbm.at[pl.ds(row0 + c * _CHUNK, _CHUNK), :]

    def out_slice(c):
      return out_hbm.at[pl.ds(row0 + c * _CHUNK, _CHUNK), :]

    def process(c, b, first_round):
      pltpu.make_async_copy(in_slice(c), in_v[b], in_sem[b]).wait()
      if not first_round:
        pltpu.make_async_copy(out_v[b], out_slice(c), out_sem[b]).wait()

      @plsc.parallel_loop(0, _JVECS, unroll=4)
      def j_body(j):
        idx = perm_v[pl.ds(j * _L, _L)]
        for r in range(_CHUNK):
          row_idx = jnp.full((_L,), r, jnp.int32)
          vals = plsc.load_gather(in_v[b], [row_idx, idx])
          out_v[b][r, pl.ds(j * _L, _L)] = vals

      pltpu.async_copy(out_v[b], out_slice(c), out_sem[b])

      @pl.when(c + _NBUF < _NCHUNKS)
      def _():
        pltpu.async_copy(in_slice(c + _NBUF), in_v[b], in_sem[b])

    for b in range(_NBUF):
      pltpu.async_copy(in_slice(b), in_v[b], in_sem[b])

    for b in range(_NBUF):
      process(b, b, True)

    def grp_body(g, carry):
      for b in range(_NBUF):
        c = (g + 1) * _NBUF + b
        process(c, b, False)
      return carry

    lax.fori_loop(0, _NGRP, grp_body, 0)

    for p in range(_NPEEL):
      c = _NBUF * (1 + _NGRP) + p
      process(c, c % _NBUF, False)

    for c in range(_NCHUNKS - _NBUF, _NCHUNKS):
      b = c % _NBUF
      pltpu.make_async_copy(out_v[b], out_slice(c), out_sem[b]).wait()

  return k(x, perm)


def kernel(x, permutation):
  return _sc_shuffle(x, permutation.astype(jnp.int32))


# speedup vs baseline: 5.0575x; 1.0084x over previous
import functools

import jax
import jax.numpy as jnp
from jax import lax
from jax.experimental import pallas as pl
from jax.experimental.pallas import tpu as pltpu
from jax.experimental.pallas import tpu_sc as plsc

_NC = 2
_NS = 16
_L = 16

_BATCH = 16384
_DIM = 2048
_NW = _NC * _NS
_ROWS_PER_W = _BATCH // _NW
_CHUNK = 8
_NCHUNKS = _ROWS_PER_W // _CHUNK
_NBUF = 3
_NGRP = (_NCHUNKS - _NBUF) // _NBUF
_NPEEL = _NCHUNKS - _NBUF * (1 + _NGRP)
_JVECS = _DIM // _L


def _sc_shuffle(x, perm):
  mesh = plsc.VectorSubcoreMesh(core_axis_name="c", subcore_axis_name="s")

  @functools.partial(
      pl.kernel,
      out_type=jax.ShapeDtypeStruct((_BATCH, _DIM), jnp.float32),
      mesh=mesh,
      scratch_types=[
          pltpu.VMEM((_DIM,), jnp.int32),
          [pltpu.VMEM((_CHUNK, _DIM), jnp.float32) for _ in range(_NBUF)],
          [pltpu.VMEM((_CHUNK, _DIM), jnp.float32) for _ in range(_NBUF)],
          [pltpu.SemaphoreType.DMA for _ in range(_NBUF)],
          [pltpu.SemaphoreType.DMA for _ in range(_NBUF)],
          pltpu.SemaphoreType.DMA,
      ],
      compiler_params=pltpu.CompilerParams(needs_layout_passes=False),
  )
  def k(x_hbm, perm_hbm, out_hbm, perm_v, in_v, out_v, in_sem, out_sem,
        perm_sem):
    wid = lax.axis_index("s") * _NC + lax.axis_index("c")
    perm_copy = pltpu.async_copy(perm_hbm, perm_v, perm_sem)
    row0 = wid * _ROWS_PER_W

    def in_slice(c):
      return x_hbm.at[pl.ds(row0 + c * _CHUNK, _CHUNK), :]

    def out_slice(c):
      return out_hbm.at[pl.ds(row0 + c * _CHUNK, _CHUNK), :]

    def process(c, b, first_round):
      pltpu.make_async_copy(in_slice(c), in_v[b], in_sem[b]).wait()
      if not first_round:
        pltpu.make_async_copy(out_v[b], out_slice(c), out_sem[b]).wait()

      @plsc.parallel_loop(0, _JVECS, unroll=4)
      def j_body(j):
        idx = perm_v[pl.ds(j * _L, _L)]
        for r in range(_CHUNK):
          row_idx = jnp.full((_L,), r, jnp.int32)
          vals = plsc.load_gather(in_v[b], [row_idx, idx])
          out_v[b][r, pl.ds(j * _L, _L)] = vals

      pltpu.async_copy(out_v[b], out_slice(c), out_sem[b])

      @pl.when(c + _NBUF < _NCHUNKS)
      def _():
        pltpu.async_copy(in_slice(c + _NBUF), in_v[b], in_sem[b])

    for b in range(_NBUF):
      pltpu.async_copy(in_slice(b), in_v[b], in_sem[b])
    perm_copy.wait()

    for b in range(_NBUF):
      process(b, b, True)

    def grp_body(g, carry):
      for b in range(_NBUF):
        c = (g + 1) * _NBUF + b
        process(c, b, False)
      return carry

    lax.fori_loop(0, _NGRP, grp_body, 0)

    for p in range(_NPEEL):
      c = _NBUF * (1 + _NGRP) + p
      process(c, c % _NBUF, False)

    for c in range(_NCHUNKS - _NBUF, _NCHUNKS):
      b = c % _NBUF
      pltpu.make_async_copy(out_v[b], out_slice(c), out_sem[b]).wait()

  return k(x, perm)


def kernel(x, permutation):
  return _sc_shuffle(x, permutation.astype(jnp.int32))
